# Initial kernel scaffold; baseline (speedup 1.0000x reference)
import jax
import jax.numpy as jnp
from jax import lax
from jax.experimental import pallas as pl
from jax.experimental.pallas import tpu as pltpu
from jax.experimental.pallas import tpu_sc as plsc

N_NODES = 10000
N_EDGES = 320000
D_IN = 128
D_HID = 16
D_OUT = 128

NC, NS = 2, 16
NW = NC * NS
EPW = N_EDGES // NW
CHUNK = 2000
NCHUNKS = EPW // CHUNK
RPT = N_NODES // NS

_mesh = plsc.VectorSubcoreMesh(
    core_axis_name="c", subcore_axis_name="s", num_cores=NC, num_subcores=NS
)


def _seg_sum_builder(with_counts):
  n_out = 2 if with_counts else 1
  out_type = [jax.ShapeDtypeStruct((NC, N_NODES, D_HID), jnp.float32)] * n_out
  scratch = [
      pltpu.VMEM((CHUNK,), jnp.int32),
      pltpu.VMEM((CHUNK,), jnp.int32),
      pltpu.VMEM((CHUNK, D_HID), jnp.float32),
      pltpu.VMEM_SHARED((N_NODES, D_HID), jnp.float32),
      pltpu.SemaphoreType.DMA,
  ]
  if with_counts:
    scratch += [
        pltpu.VMEM((CHUNK, D_HID), jnp.float32),
        pltpu.VMEM_SHARED((N_NODES, D_HID), jnp.float32),
    ]

  def body(*refs):
    if with_counts:
      (tbl, src_hbm, dst_hbm, zeros_hbm, ones_hbm,
       sums_out, cnts_out, src_v, dst_v, rows_v, acc_sh, sem,
       ones_v, cacc_sh) = refs
    else:
      (tbl, src_hbm, dst_hbm, zeros_hbm,
       sums_out, src_v, dst_v, rows_v, acc_sh, sem) = refs
    c = lax.axis_index("c")
    s = lax.axis_index("s")
    wid = s * NC + c
    rs = s * RPT
    pltpu.sync_copy(zeros_hbm.at[pl.ds(rs, RPT)], acc_sh.at[pl.ds(rs, RPT)])
    if with_counts:
      pltpu.sync_copy(zeros_hbm.at[pl.ds(rs, RPT)], cacc_sh.at[pl.ds(rs, RPT)])
      pltpu.sync_copy(ones_hbm, ones_v)
    plsc.subcore_barrier()
    for i in range(NCHUNKS):
      b = wid * EPW + i * CHUNK
      pltpu.sync_copy(src_hbm.at[pl.ds(b, CHUNK)], src_v)
      pltpu.sync_copy(dst_hbm.at[pl.ds(b, CHUNK)], dst_v)
      pltpu.async_copy(tbl.at[src_v], rows_v, sem).wait()
      pltpu.sync_copy(rows_v, acc_sh.at[dst_v], add=True)
      if with_counts:
        pltpu.sync_copy(ones_v, cacc_sh.at[dst_v], add=True)
    plsc.subcore_barrier()
    pltpu.sync_copy(acc_sh.at[pl.ds(rs, RPT)],
                    sums_out.at[c, pl.ds(rs, RPT)])
    if with_counts:
      pltpu.sync_copy(cacc_sh.at[pl.ds(rs, RPT)],
                      cnts_out.at[c, pl.ds(rs, RPT)])

  return pl.kernel(body, out_type=out_type, mesh=_mesh, scratch_types=scratch,
                   name="seg_sum_cnt" if with_counts else "seg_sum")


_seg_sum_cnt = _seg_sum_builder(True)
_seg_sum = _seg_sum_builder(False)


def _proj_body(x_ref, w_ref, o_ref):
  o_ref[...] = jnp.dot(x_ref[...], w_ref[...],
                       preferred_element_type=jnp.float32)


_proj = pl.pallas_call(
    _proj_body,
    out_shape=jax.ShapeDtypeStruct((N_NODES, 2 * D_HID), jnp.float32),
)


def _post1_body(s0, s1, c0, c1, r, b, h_out, cnt_out):
  cnt = jnp.maximum(c0[...] + c1[...], 1.0)
  t = (s0[...] + s1[...]) / cnt + b[...] + r[...]
  nrm = jnp.sqrt(jnp.sum(t * t, axis=1, keepdims=True))
  t = t / jnp.maximum(nrm, 1e-12)
  h_out[...] = jnp.maximum(t, 0.0)
  cnt_out[...] = cnt


_post1 = pl.pallas_call(
    _post1_body,
    out_shape=[jax.ShapeDtypeStruct((N_NODES, D_HID), jnp.float32),
               jax.ShapeDtypeStruct((N_NODES, D_HID), jnp.float32)],
)


def _post2_body(s0, s1, cnt, h, w2, b2, o_ref):
  mean2 = (s0[...] + s1[...]) / cnt[...]
  g = jnp.concatenate([mean2, h[...]], axis=1)
  t = jnp.dot(g, w2[...], preferred_element_type=jnp.float32) + b2[...]
  nrm = jnp.sqrt(jnp.sum(t * t, axis=1, keepdims=True))
  o_ref[...] = t / jnp.maximum(nrm, 1e-12)


_post2 = pl.pallas_call(
    _post2_body,
    out_shape=jax.ShapeDtypeStruct((N_NODES, D_OUT), jnp.float32),
)


@jax.jit
def kernel(x, edge_index, Wl1, Wr1, b1, Wl2, Wr2, b2):
  src = edge_index[0].astype(jnp.int32)
  dst = edge_index[1].astype(jnp.int32)
  zeros = jnp.zeros((N_NODES, D_HID), jnp.float32)
  ones = jnp.ones((CHUNK, D_HID), jnp.float32)

  w1 = jnp.concatenate([Wl1, Wr1], axis=1)
  pr = _proj(x, w1)
  p = pr[:, :D_HID]
  r = pr[:, D_HID:]

  sums1, cnts1 = _seg_sum_cnt(p, src, dst, zeros, ones)
  h, cnt = _post1(sums1[0], sums1[1], cnts1[0], cnts1[1], r,
                  b1.reshape(1, D_HID))

  (sums2,) = _seg_sum(h, src, dst, zeros)
  w2 = jnp.concatenate([Wl2, Wr2], axis=0)
  return _post2(sums2[0], sums2[1], cnt, h, w2, b2.reshape(1, D_OUT))


# speedup vs baseline: 19.0629x; 19.0629x over previous
import functools

import jax
import jax.numpy as jnp
from jax import lax
from jax.experimental import pallas as pl
from jax.experimental.pallas import tpu as pltpu
from jax.experimental.pallas import tpu_sc as plsc

N_NODES = 10000
N_EDGES = 320000
D_IN = 128
D_HID = 16
D_OUT = 128

NC, NS = 2, 16
NW = NC * NS
EPW = N_EDGES // NW
CHUNK = 2000
NCHUNKS = EPW // CHUNK
N_PAD = 10240
RPT = N_PAD // NS

@functools.cache
def _seg_sum_builder(with_counts):
  n_out = 2 if with_counts else 1
  out_type = [jax.ShapeDtypeStruct((NC, N_PAD, D_HID), jnp.float32)] * n_out
  scratch = [
      pltpu.VMEM((CHUNK,), jnp.int32),
      pltpu.VMEM((CHUNK,), jnp.int32),
      pltpu.VMEM((CHUNK, D_HID), jnp.float32),
      pltpu.VMEM_SHARED((N_PAD, D_HID), jnp.float32),
      pltpu.SemaphoreType.DMA,
  ]
  if with_counts:
    scratch += [
        pltpu.VMEM((CHUNK, D_HID), jnp.float32),
        pltpu.VMEM_SHARED((N_PAD, D_HID), jnp.float32),
    ]

  def body(*refs):
    if with_counts:
      (tbl, src_hbm, dst_hbm, zeros_hbm, ones_hbm,
       sums_out, cnts_out, src_v, dst_v, rows_v, acc_sh, sem,
       ones_v, cacc_sh) = refs
    else:
      (tbl, src_hbm, dst_hbm, zeros_hbm,
       sums_out, src_v, dst_v, rows_v, acc_sh, sem) = refs
    c = lax.axis_index("c")
    s = lax.axis_index("s")
    wid = s * NC + c
    rs = s * RPT
    pltpu.sync_copy(zeros_hbm.at[pl.ds(rs, RPT)], acc_sh.at[pl.ds(rs, RPT)])
    if with_counts:
      pltpu.sync_copy(zeros_hbm.at[pl.ds(rs, RPT)], cacc_sh.at[pl.ds(rs, RPT)])
      pltpu.sync_copy(ones_hbm, ones_v)
    plsc.subcore_barrier()
    for i in range(NCHUNKS):
      b = wid * EPW + i * CHUNK
      pltpu.sync_copy(src_hbm.at[pl.ds(b, CHUNK)], src_v)
      pltpu.sync_copy(dst_hbm.at[pl.ds(b, CHUNK)], dst_v)
      pltpu.async_copy(tbl.at[src_v], rows_v, sem).wait()
      pltpu.sync_copy(rows_v, acc_sh.at[dst_v], add=True)
      if with_counts:
        pltpu.sync_copy(ones_v, cacc_sh.at[dst_v], add=True)
    plsc.subcore_barrier()
    pltpu.sync_copy(acc_sh.at[pl.ds(rs, RPT)],
                    sums_out.at[c, pl.ds(rs, RPT)])
    if with_counts:
      pltpu.sync_copy(cacc_sh.at[pl.ds(rs, RPT)],
                      cnts_out.at[c, pl.ds(rs, RPT)])

  mesh = plsc.VectorSubcoreMesh(
      core_axis_name="c", subcore_axis_name="s", num_cores=NC, num_subcores=NS
  )
  return pl.kernel(body, out_type=out_type, mesh=mesh, scratch_types=scratch,
                   compiler_params=pltpu.CompilerParams(
                       use_tc_tiling_on_sc=False),
                   name="seg_sum_cnt" if with_counts else "seg_sum")


def _proj_body(x_ref, w_ref, o_ref):
  o_ref[...] = jnp.dot(x_ref[...], w_ref[...],
                       preferred_element_type=jnp.float32)


_proj = pl.pallas_call(
    _proj_body,
    out_shape=jax.ShapeDtypeStruct((N_NODES, 2 * D_HID), jnp.float32),
)


def _post1_body(s0, s1, c0, c1, r, b, h_out, cnt_out):
  cnt = jnp.maximum(c0[...] + c1[...], 1.0)
  t = (s0[...] + s1[...]) / cnt + b[...] + r[...]
  nrm = jnp.sqrt(jnp.sum(t * t, axis=1, keepdims=True))
  t = t / jnp.maximum(nrm, 1e-12)
  h_out[...] = jnp.maximum(t, 0.0)
  cnt_out[...] = cnt


_post1 = pl.pallas_call(
    _post1_body,
    out_shape=[jax.ShapeDtypeStruct((N_NODES, D_HID), jnp.float32),
               jax.ShapeDtypeStruct((N_NODES, D_HID), jnp.float32)],
)


def _post2_body(s0, s1, cnt, h, w2, b2, o_ref):
  mean2 = (s0[...] + s1[...]) / cnt[...]
  g = jnp.concatenate([mean2, h[...]], axis=1)
  t = jnp.dot(g, w2[...], preferred_element_type=jnp.float32) + b2[...]
  nrm = jnp.sqrt(jnp.sum(t * t, axis=1, keepdims=True))
  o_ref[...] = t / jnp.maximum(nrm, 1e-12)


_post2 = pl.pallas_call(
    _post2_body,
    out_shape=jax.ShapeDtypeStruct((N_NODES, D_OUT), jnp.float32),
)


@jax.jit
def kernel(x, edge_index, Wl1, Wr1, b1, Wl2, Wr2, b2):
  src = edge_index[0].astype(jnp.int32)
  dst = edge_index[1].astype(jnp.int32)
  zeros = jnp.zeros((N_PAD, D_HID), jnp.float32)
  ones = jnp.ones((CHUNK, D_HID), jnp.float32)

  w1 = jnp.concatenate([Wl1, Wr1], axis=1)
  pr = _proj(x, w1)
  p = pr[:, :D_HID]
  r = pr[:, D_HID:]

  sums1, cnts1 = _seg_sum_builder(True)(p, src, dst, zeros, ones)
  h, cnt = _post1(sums1[0, :N_NODES], sums1[1, :N_NODES],
                  cnts1[0, :N_NODES], cnts1[1, :N_NODES], r,
                  b1.reshape(1, D_HID))

  (sums2,) = _seg_sum_builder(False)(h, src, dst, zeros)
  w2 = jnp.concatenate([Wl2, Wr2], axis=0)
  return _post2(sums2[0, :N_NODES], sums2[1, :N_NODES], cnt, h, w2,
                b2.reshape(1, D_OUT))


# speedup vs baseline: 22.4525x; 1.1778x over previous
import functools

import jax
import jax.numpy as jnp
from jax import lax
from jax.experimental import pallas as pl
from jax.experimental.pallas import tpu as pltpu
from jax.experimental.pallas import tpu_sc as plsc

N_NODES = 10000
N_EDGES = 320000
D_IN = 128
D_HID = 16
D_OUT = 128

NC, NS = 2, 16
NW = NC * NS
EPW = N_EDGES // NW
CHUNK = 2000
NCHUNKS = EPW // CHUNK
N_PAD = 10240
RPT = N_PAD // NS

@functools.cache
def _seg_sum_builder(with_counts):
  n_out = 2 if with_counts else 1
  out_type = [jax.ShapeDtypeStruct((NC, N_PAD, D_HID), jnp.float32)] * n_out
  scratch = [
      pltpu.VMEM((CHUNK,), jnp.int32),
      pltpu.VMEM((CHUNK,), jnp.int32),
      pltpu.VMEM((CHUNK, D_HID), jnp.float32),
      pltpu.VMEM_SHARED((N_PAD, D_HID), jnp.float32),
      pltpu.SemaphoreType.DMA,
  ]
  if with_counts:
    scratch += [
        pltpu.VMEM((CHUNK, D_HID), jnp.float32),
        pltpu.VMEM_SHARED((N_PAD, D_HID), jnp.float32),
    ]

  def body(*refs):
    if with_counts:
      (tbl, ei_hbm, zeros_hbm, ones_hbm,
       sums_out, cnts_out, src_v, dst_v, rows_v, acc_sh, sem,
       ones_v, cacc_sh) = refs
    else:
      (tbl, ei_hbm, zeros_hbm,
       sums_out, src_v, dst_v, rows_v, acc_sh, sem) = refs
    c = lax.axis_index("c")
    s = lax.axis_index("s")
    wid = s * NC + c
    rs = s * RPT
    pltpu.sync_copy(zeros_hbm.at[pl.ds(rs, RPT)], acc_sh.at[pl.ds(rs, RPT)])
    if with_counts:
      pltpu.sync_copy(zeros_hbm.at[pl.ds(rs, RPT)], cacc_sh.at[pl.ds(rs, RPT)])
      pltpu.sync_copy(ones_hbm, ones_v)
    plsc.subcore_barrier()
    for i in range(NCHUNKS):
      b = wid * EPW + i * CHUNK
      pltpu.sync_copy(ei_hbm.at[0, pl.ds(b, CHUNK)], src_v)
      pltpu.sync_copy(ei_hbm.at[1, pl.ds(b, CHUNK)], dst_v)
      pltpu.async_copy(tbl.at[src_v], rows_v, sem).wait()
      pltpu.sync_copy(rows_v, acc_sh.at[dst_v], add=True)
      if with_counts:
        pltpu.sync_copy(ones_v, cacc_sh.at[dst_v], add=True)
    plsc.subcore_barrier()
    pltpu.sync_copy(acc_sh.at[pl.ds(rs, RPT)],
                    sums_out.at[c, pl.ds(rs, RPT)])
    if with_counts:
      pltpu.sync_copy(cacc_sh.at[pl.ds(rs, RPT)],
                      cnts_out.at[c, pl.ds(rs, RPT)])

  mesh = plsc.VectorSubcoreMesh(
      core_axis_name="c", subcore_axis_name="s", num_cores=NC, num_subcores=NS
  )
  return pl.kernel(body, out_type=out_type, mesh=mesh, scratch_types=scratch,
                   compiler_params=pltpu.CompilerParams(
                       use_tc_tiling_on_sc=False),
                   name="seg_sum_cnt" if with_counts else "seg_sum")


def _proj_body(x_ref, wl_ref, wr_ref, p_ref, r_ref):
  x = x_ref[...]
  p_ref[...] = jnp.dot(x, wl_ref[...], preferred_element_type=jnp.float32)
  r_ref[...] = jnp.dot(x, wr_ref[...], preferred_element_type=jnp.float32)


_proj = pl.pallas_call(
    _proj_body,
    out_shape=[jax.ShapeDtypeStruct((N_NODES, D_HID), jnp.float32),
               jax.ShapeDtypeStruct((N_NODES, D_HID), jnp.float32)],
)


def _post1_body(sums, cnts, r, b, h_out, cnt_out):
  s = sums[0, :N_NODES] + sums[1, :N_NODES]
  cnt = jnp.maximum(cnts[0, :N_NODES] + cnts[1, :N_NODES], 1.0)
  t = s / cnt + b[...] + r[...]
  nrm = jnp.sqrt(jnp.sum(t * t, axis=1, keepdims=True))
  t = t / jnp.maximum(nrm, 1e-12)
  h_out[...] = jnp.maximum(t, 0.0)
  cnt_out[...] = cnt


_post1 = pl.pallas_call(
    _post1_body,
    out_shape=[jax.ShapeDtypeStruct((N_NODES, D_HID), jnp.float32),
               jax.ShapeDtypeStruct((N_NODES, D_HID), jnp.float32)],
)


def _post2_body(sums, cnt, h, wl, wr, b2, o_ref):
  mean2 = (sums[0, :N_NODES] + sums[1, :N_NODES]) / cnt[...]
  t = (jnp.dot(mean2, wl[...], preferred_element_type=jnp.float32)
       + jnp.dot(h[...], wr[...], preferred_element_type=jnp.float32)
       + b2[...])
  nrm = jnp.sqrt(jnp.sum(t * t, axis=1, keepdims=True))
  o_ref[...] = t / jnp.maximum(nrm, 1e-12)


_post2 = pl.pallas_call(
    _post2_body,
    out_shape=jax.ShapeDtypeStruct((N_NODES, D_OUT), jnp.float32),
)


@jax.jit
def kernel(x, edge_index, Wl1, Wr1, b1, Wl2, Wr2, b2):
  ei = edge_index.astype(jnp.int32)
  zeros = jnp.zeros((N_PAD, D_HID), jnp.float32)
  ones = jnp.ones((CHUNK, D_HID), jnp.float32)

  p, r = _proj(x, Wl1, Wr1)
  sums1, cnts1 = _seg_sum_builder(True)(p, ei, zeros, ones)
  h, cnt = _post1(sums1, cnts1, r, b1.reshape(1, D_HID))

  (sums2,) = _seg_sum_builder(False)(h, ei, zeros)
  return _post2(sums2, cnt, h, Wl2, Wr2, b2.reshape(1, D_OUT))


# speedup vs baseline: 25.4808x; 1.1349x over previous
import functools

import jax
import jax.numpy as jnp
from jax import lax
from jax.experimental import pallas as pl
from jax.experimental.pallas import tpu as pltpu
from jax.experimental.pallas import tpu_sc as plsc

N_NODES = 10000
N_EDGES = 320000
D_IN = 128
D_HID = 16
D_OUT = 128

NC, NS = 2, 16
NW = NC * NS
EPW = N_EDGES // NW
CHUNK = 2000
NCHUNKS = EPW // CHUNK
N_PAD = 10240
RPT = N_PAD // NS

@functools.cache
def _seg_sum_builder(with_counts):
  n_out = 2 if with_counts else 1
  out_type = [jax.ShapeDtypeStruct((NC, N_PAD, D_HID), jnp.float32)] * n_out
  scratch = [
      [pltpu.VMEM((CHUNK,), jnp.int32)] * 2,
      [pltpu.VMEM((CHUNK,), jnp.int32)] * 2,
      [pltpu.VMEM((CHUNK, D_HID), jnp.float32)] * 2,
      pltpu.VMEM_SHARED((N_PAD, D_HID), jnp.float32),
      [pltpu.SemaphoreType.DMA] * 2,
  ]
  if with_counts:
    scratch += [
        pltpu.VMEM((CHUNK, D_HID), jnp.float32),
        pltpu.VMEM_SHARED((N_PAD, D_HID), jnp.float32),
    ]

  def body(*refs):
    if with_counts:
      (tbl, ei_hbm, zeros_hbm, ones_hbm,
       sums_out, cnts_out, src_v, dst_v, rows_v, acc_sh, sem,
       ones_v, cacc_sh) = refs
    else:
      (tbl, ei_hbm, zeros_hbm,
       sums_out, src_v, dst_v, rows_v, acc_sh, sem) = refs
    c = lax.axis_index("c")
    s = lax.axis_index("s")
    wid = s * NC + c
    rs = s * RPT
    pltpu.sync_copy(zeros_hbm.at[pl.ds(rs, RPT)], acc_sh.at[pl.ds(rs, RPT)])
    if with_counts:
      pltpu.sync_copy(zeros_hbm.at[pl.ds(rs, RPT)], cacc_sh.at[pl.ds(rs, RPT)])
      pltpu.sync_copy(ones_hbm, ones_v)
    plsc.subcore_barrier()

    def start_chunk(i):
      b = wid * EPW + i * CHUNK
      k = i % 2
      pltpu.sync_copy(ei_hbm.at[0, pl.ds(b, CHUNK)], src_v[k])
      pltpu.sync_copy(ei_hbm.at[1, pl.ds(b, CHUNK)], dst_v[k])
      return pltpu.async_copy(tbl.at[src_v[k]], rows_v[k], sem[k])

    gather = start_chunk(0)
    for i in range(NCHUNKS):
      k = i % 2
      nxt = start_chunk(i + 1) if i + 1 < NCHUNKS else None
      gather.wait()
      pltpu.sync_copy(rows_v[k], acc_sh.at[dst_v[k]], add=True)
      if with_counts:
        pltpu.sync_copy(ones_v, cacc_sh.at[dst_v[k]], add=True)
      gather = nxt
    plsc.subcore_barrier()
    pltpu.sync_copy(acc_sh.at[pl.ds(rs, RPT)],
                    sums_out.at[c, pl.ds(rs, RPT)])
    if with_counts:
      pltpu.sync_copy(cacc_sh.at[pl.ds(rs, RPT)],
                      cnts_out.at[c, pl.ds(rs, RPT)])

  mesh = plsc.VectorSubcoreMesh(
      core_axis_name="c", subcore_axis_name="s", num_cores=NC, num_subcores=NS
  )
  return pl.kernel(body, out_type=out_type, mesh=mesh, scratch_types=scratch,
                   compiler_params=pltpu.CompilerParams(
                       use_tc_tiling_on_sc=False),
                   name="seg_sum_cnt" if with_counts else "seg_sum")


def _proj_body(x_ref, wl_ref, wr_ref, p_ref, r_ref):
  x = x_ref[...]
  p_ref[...] = jnp.dot(x, wl_ref[...], preferred_element_type=jnp.float32)
  r_ref[...] = jnp.dot(x, wr_ref[...], preferred_element_type=jnp.float32)


_proj = pl.pallas_call(
    _proj_body,
    out_shape=[jax.ShapeDtypeStruct((N_NODES, D_HID), jnp.float32),
               jax.ShapeDtypeStruct((N_NODES, D_HID), jnp.float32)],
)


def _post1_body(sums, cnts, r, b, h_out, cnt_out):
  s = sums[0, :N_NODES] + sums[1, :N_NODES]
  cnt = jnp.maximum(cnts[0, :N_NODES] + cnts[1, :N_NODES], 1.0)
  t = s / cnt + b[...] + r[...]
  nrm = jnp.sqrt(jnp.sum(t * t, axis=1, keepdims=True))
  t = t / jnp.maximum(nrm, 1e-12)
  h_out[...] = jnp.maximum(t, 0.0)
  cnt_out[...] = cnt


_post1 = pl.pallas_call(
    _post1_body,
    out_shape=[jax.ShapeDtypeStruct((N_NODES, D_HID), jnp.float32),
               jax.ShapeDtypeStruct((N_NODES, D_HID), jnp.float32)],
)


def _post2_body(sums, cnt, h, wl, wr, b2, o_ref):
  mean2 = (sums[0, :N_NODES] + sums[1, :N_NODES]) / cnt[...]
  t = (jnp.dot(mean2, wl[...], preferred_element_type=jnp.float32)
       + jnp.dot(h[...], wr[...], preferred_element_type=jnp.float32)
       + b2[...])
  nrm = jnp.sqrt(jnp.sum(t * t, axis=1, keepdims=True))
  o_ref[...] = t / jnp.maximum(nrm, 1e-12)


_post2 = pl.pallas_call(
    _post2_body,
    out_shape=jax.ShapeDtypeStruct((N_NODES, D_OUT), jnp.float32),
)


@jax.jit
def kernel(x, edge_index, Wl1, Wr1, b1, Wl2, Wr2, b2):
  ei = edge_index.astype(jnp.int32)
  zeros = jnp.zeros((N_PAD, D_HID), jnp.float32)
  ones = jnp.ones((CHUNK, D_HID), jnp.float32)

  p, r = _proj(x, Wl1, Wr1)
  sums1, cnts1 = _seg_sum_builder(True)(p, ei, zeros, ones)
  h, cnt = _post1(sums1, cnts1, r, b1.reshape(1, D_HID))

  (sums2,) = _seg_sum_builder(False)(h, ei, zeros)
  return _post2(sums2, cnt, h, Wl2, Wr2, b2.reshape(1, D_OUT))


# speedup vs baseline: 26.6008x; 1.0440x over previous
import functools

import jax
import jax.numpy as jnp
from jax import lax
from jax.experimental import pallas as pl
from jax.experimental.pallas import tpu as pltpu
from jax.experimental.pallas import tpu_sc as plsc

N_NODES = 10000
N_EDGES = 320000
D_IN = 128
D_HID = 16
D_OUT = 128

NC, NS = 2, 16
NW = NC * NS
EPW = N_EDGES // NW
CHUNK = 2000
NCHUNKS = EPW // CHUNK
N_PAD = 10240
RPT = N_PAD // NS

@functools.cache
def _seg_sum_builder(with_counts):
  n_out = 2 if with_counts else 1
  out_type = [jax.ShapeDtypeStruct((NC, N_PAD, D_HID), jnp.float32)] * n_out
  scratch = [
      [pltpu.VMEM((CHUNK,), jnp.int32)] * 2,
      [pltpu.VMEM((CHUNK,), jnp.int32)] * 2,
      [pltpu.VMEM((CHUNK, D_HID), jnp.float32)] * 2,
      pltpu.VMEM_SHARED((N_PAD, D_HID), jnp.float32),
      [pltpu.SemaphoreType.DMA] * 2,
  ]
  if with_counts:
    scratch += [
        pltpu.VMEM((CHUNK, D_HID), jnp.float32),
        pltpu.VMEM_SHARED((N_PAD, D_HID), jnp.float32),
    ]

  def body(*refs):
    if with_counts:
      (tbl, ei_hbm, zeros_hbm, ones_hbm,
       sums_out, cnts_out, src_v, dst_v, rows_v, acc_sh, sem,
       ones_v, cacc_sh) = refs
    else:
      (tbl, ei_hbm, zeros_hbm,
       sums_out, src_v, dst_v, rows_v, acc_sh, sem) = refs
    c = lax.axis_index("c")
    s = lax.axis_index("s")
    wid = s * NC + c
    rs = s * RPT
    pltpu.sync_copy(zeros_hbm.at[pl.ds(rs, RPT)], acc_sh.at[pl.ds(rs, RPT)])
    if with_counts:
      pltpu.sync_copy(zeros_hbm.at[pl.ds(rs, RPT)], cacc_sh.at[pl.ds(rs, RPT)])
      pltpu.sync_copy(ones_hbm, ones_v)
    plsc.subcore_barrier()

    def start_chunk(i):
      b = wid * EPW + i * CHUNK
      k = i % 2
      pltpu.sync_copy(ei_hbm.at[0, pl.ds(b, CHUNK)], src_v[k])
      pltpu.sync_copy(ei_hbm.at[1, pl.ds(b, CHUNK)], dst_v[k])
      return pltpu.async_copy(tbl.at[src_v[k]], rows_v[k], sem[k])

    gather = start_chunk(0)
    for i in range(NCHUNKS):
      k = i % 2
      nxt = start_chunk(i + 1) if i + 1 < NCHUNKS else None
      gather.wait()
      pltpu.sync_copy(rows_v[k], acc_sh.at[dst_v[k]], add=True)
      if with_counts:
        pltpu.sync_copy(ones_v, cacc_sh.at[dst_v[k]], add=True)
      gather = nxt
    plsc.subcore_barrier()
    pltpu.sync_copy(acc_sh.at[pl.ds(rs, RPT)],
                    sums_out.at[c, pl.ds(rs, RPT)])
    if with_counts:
      pltpu.sync_copy(cacc_sh.at[pl.ds(rs, RPT)],
                      cnts_out.at[c, pl.ds(rs, RPT)])

  mesh = plsc.VectorSubcoreMesh(
      core_axis_name="c", subcore_axis_name="s", num_cores=NC, num_subcores=NS
  )
  return pl.kernel(body, out_type=out_type, mesh=mesh, scratch_types=scratch,
                   compiler_params=pltpu.CompilerParams(
                       use_tc_tiling_on_sc=False),
                   name="seg_sum_cnt" if with_counts else "seg_sum")


HS = RPT // 2


def _newton_rsqrt(v):
  i = lax.bitcast_convert_type(v, jnp.int32)
  y = lax.bitcast_convert_type(jnp.int32(0x5F3759DF) - (i >> 1), jnp.float32)
  for _ in range(3):
    y = y * (1.5 - 0.5 * v * y * y)
  return y


_GDN = lax.GatherDimensionNumbers(
    offset_dims=(), collapsed_slice_dims=(0,), start_index_map=(0,))


def _lane_perm(v, idx):
  return lax.gather(v, idx[:, None], _GDN, slice_sizes=(1,),
                    mode=lax.GatherScatterMode.PROMISE_IN_BOUNDS)


def _lane_sum(v):
  lanes = lax.iota(jnp.int32, 16)
  for sh in (8, 4, 2, 1):
    v = v + _lane_perm(v, lanes ^ sh)
  return v


@functools.cache
def _seg2_builder():
  out_type = [
      jax.ShapeDtypeStruct((NC, N_PAD, D_HID), jnp.float32),
      jax.ShapeDtypeStruct((NC, N_PAD, D_HID), jnp.float32),
  ]
  scratch = [
      [pltpu.VMEM((CHUNK,), jnp.int32)] * 2,
      [pltpu.VMEM((CHUNK,), jnp.int32)] * 2,
      [pltpu.VMEM((CHUNK, D_HID), jnp.float32)] * 2,
      pltpu.VMEM_SHARED((N_PAD, D_HID), jnp.float32),
      [pltpu.SemaphoreType.DMA] * 2,
      [pltpu.VMEM((HS, D_HID), jnp.float32)] * 6,
      pltpu.VMEM((D_HID,), jnp.float32),
  ]

  def body(sums1, cnts1, r_hbm, b1_hbm, ei_hbm, zeros_hbm,
           sums_out, h_out, src_v, dst_v, rows_v, acc_sh, sem, ep, b_v):
    c = lax.axis_index("c")
    s = lax.axis_index("s")
    wid = s * NC + c
    rs = s * RPT
    pltpu.sync_copy(zeros_hbm.at[pl.ds(rs, RPT)], acc_sh.at[pl.ds(rs, RPT)])
    pltpu.sync_copy(b1_hbm, b_v)
    bias = b_v[...]
    sA, sB, cA, cB, rbuf, hbuf = ep

    for half in range(2):
      r0 = rs + half * HS
      pltpu.sync_copy(sums1.at[0, pl.ds(r0, HS)], sA)
      pltpu.sync_copy(sums1.at[1, pl.ds(r0, HS)], sB)
      pltpu.sync_copy(cnts1.at[0, pl.ds(r0, HS)], cA)
      pltpu.sync_copy(cnts1.at[1, pl.ds(r0, HS)], cB)
      pltpu.sync_copy(r_hbm.at[pl.ds(r0, HS)], rbuf)

      def row(j, _):
        cnt = jnp.maximum(cA[j, :] + cB[j, :], 1.0)
        t = (sA[j, :] + sB[j, :]) / cnt + bias + rbuf[j, :]
        n2 = jnp.maximum(_lane_sum(t * t), 1e-24)
        hbuf[j, :] = jnp.maximum(t * _newton_rsqrt(n2), 0.0)
        return 0

      lax.fori_loop(0, HS, row, 0)
      pltpu.sync_copy(hbuf, h_out.at[c, pl.ds(r0, HS)])
    plsc.subcore_barrier()

    def start_chunk(i):
      b = wid * EPW + i * CHUNK
      k = i % 2
      pltpu.sync_copy(ei_hbm.at[0, pl.ds(b, CHUNK)], src_v[k])
      pltpu.sync_copy(ei_hbm.at[1, pl.ds(b, CHUNK)], dst_v[k])
      return pltpu.async_copy(h_out.at[c].at[src_v[k]], rows_v[k], sem[k])

    gather = start_chunk(0)
    for i in range(NCHUNKS):
      k = i % 2
      nxt = start_chunk(i + 1) if i + 1 < NCHUNKS else None
      gather.wait()
      pltpu.sync_copy(rows_v[k], acc_sh.at[dst_v[k]], add=True)
      gather = nxt
    plsc.subcore_barrier()
    pltpu.sync_copy(acc_sh.at[pl.ds(rs, RPT)], sums_out.at[c, pl.ds(rs, RPT)])

  mesh = plsc.VectorSubcoreMesh(
      core_axis_name="c", subcore_axis_name="s", num_cores=NC, num_subcores=NS
  )
  return pl.kernel(body, out_type=out_type, mesh=mesh, scratch_types=scratch,
                   compiler_params=pltpu.CompilerParams(
                       use_tc_tiling_on_sc=False),
                   name="seg2_fused")


def _proj_body(x_ref, wl_ref, wr_ref, p_ref, r_ref):
  x = x_ref[...]
  p_ref[...] = jnp.dot(x, wl_ref[...], preferred_element_type=jnp.float32)
  r_ref[pl.ds(0, N_NODES), :] = jnp.dot(x, wr_ref[...],
                                        preferred_element_type=jnp.float32)
  r_ref[pl.ds(N_NODES, N_PAD - N_NODES), :] = jnp.zeros(
      (N_PAD - N_NODES, D_HID), jnp.float32)


_proj = pl.pallas_call(
    _proj_body,
    out_shape=[jax.ShapeDtypeStruct((N_NODES, D_HID), jnp.float32),
               jax.ShapeDtypeStruct((N_PAD, D_HID), jnp.float32)],
)


def _post2_body(sums, cnts, h, wl, wr, b2, o_ref):
  cnt = jnp.maximum(cnts[0, :N_NODES] + cnts[1, :N_NODES], 1.0)
  mean2 = (sums[0, :N_NODES] + sums[1, :N_NODES]) / cnt
  t = (jnp.dot(mean2, wl[...], preferred_element_type=jnp.float32)
       + jnp.dot(h[0, :N_NODES], wr[...], preferred_element_type=jnp.float32)
       + b2[...])
  nrm = jnp.sqrt(jnp.sum(t * t, axis=1, keepdims=True))
  o_ref[...] = t / jnp.maximum(nrm, 1e-12)


_post2 = pl.pallas_call(
    _post2_body,
    out_shape=jax.ShapeDtypeStruct((N_NODES, D_OUT), jnp.float32),
)


@jax.jit
def kernel(x, edge_index, Wl1, Wr1, b1, Wl2, Wr2, b2):
  ei = edge_index.astype(jnp.int32)
  zeros = jnp.zeros((N_PAD, D_HID), jnp.float32)
  ones = jnp.ones((CHUNK, D_HID), jnp.float32)

  p, r = _proj(x, Wl1, Wr1)
  sums1, cnts1 = _seg_sum_builder(True)(p, ei, zeros, ones)
  sums2, h = _seg2_builder()(sums1, cnts1, r, b1, ei, zeros)
  return _post2(sums2, cnts1, h, Wl2, Wr2, b2.reshape(1, D_OUT))


# speedup vs baseline: 29.1486x; 1.0958x over previous
import functools

import jax
import jax.numpy as jnp
from jax import lax
from jax.experimental import pallas as pl
from jax.experimental.pallas import tpu as pltpu
from jax.experimental.pallas import tpu_sc as plsc

N_NODES = 10000
N_EDGES = 320000
D_IN = 128
D_HID = 16
D_OUT = 128

NC, NS = 2, 16
NW = NC * NS
EPW = N_EDGES // NW
CHUNK = 2000
NCHUNKS = EPW // CHUNK
N_PAD = 10240
RPT = N_PAD // NS

@functools.cache
def _seg_sum_builder(with_counts):
  n_out = 2 if with_counts else 1
  out_type = [jax.ShapeDtypeStruct((NC, N_PAD, D_HID), jnp.float32)] * n_out
  scratch = [
      [pltpu.VMEM((CHUNK,), jnp.int32)] * 2,
      [pltpu.VMEM((CHUNK,), jnp.int32)] * 2,
      [pltpu.VMEM((CHUNK, D_HID), jnp.float32)] * 2,
      pltpu.VMEM_SHARED((N_PAD, D_HID), jnp.float32),
      [pltpu.SemaphoreType.DMA] * 2,
  ]
  if with_counts:
    scratch += [
        pltpu.VMEM((CHUNK, D_HID), jnp.float32),
        pltpu.VMEM_SHARED((N_PAD, D_HID), jnp.float32),
    ]

  def body(*refs):
    if with_counts:
      (tbl, ei_hbm, zeros_hbm, ones_hbm,
       sums_out, cnts_out, src_v, dst_v, rows_v, acc_sh, sem,
       ones_v, cacc_sh) = refs
    else:
      (tbl, ei_hbm, zeros_hbm,
       sums_out, src_v, dst_v, rows_v, acc_sh, sem) = refs
    c = lax.axis_index("c")
    s = lax.axis_index("s")
    wid = s * NC + c
    rs = s * RPT
    pltpu.sync_copy(zeros_hbm.at[pl.ds(rs, RPT)], acc_sh.at[pl.ds(rs, RPT)])
    if with_counts:
      pltpu.sync_copy(zeros_hbm.at[pl.ds(rs, RPT)], cacc_sh.at[pl.ds(rs, RPT)])
      pltpu.sync_copy(ones_hbm, ones_v)
    plsc.subcore_barrier()

    def start_chunk(i):
      b = wid * EPW + i * CHUNK
      k = i % 2
      pltpu.sync_copy(ei_hbm.at[0, pl.ds(b, CHUNK)], src_v[k])
      pltpu.sync_copy(ei_hbm.at[1, pl.ds(b, CHUNK)], dst_v[k])
      return pltpu.async_copy(tbl.at[src_v[k]], rows_v[k], sem[k])

    gather = start_chunk(0)
    for i in range(NCHUNKS):
      k = i % 2
      nxt = start_chunk(i + 1) if i + 1 < NCHUNKS else None
      gather.wait()
      pltpu.sync_copy(rows_v[k], acc_sh.at[dst_v[k]], add=True)
      if with_counts:
        pltpu.sync_copy(ones_v, cacc_sh.at[dst_v[k]], add=True)
      gather = nxt
    plsc.subcore_barrier()
    pltpu.sync_copy(acc_sh.at[pl.ds(rs, RPT)],
                    sums_out.at[c, pl.ds(rs, RPT)])
    if with_counts:
      pltpu.sync_copy(cacc_sh.at[pl.ds(rs, RPT)],
                      cnts_out.at[c, pl.ds(rs, RPT)])

  mesh = plsc.VectorSubcoreMesh(
      core_axis_name="c", subcore_axis_name="s", num_cores=NC, num_subcores=NS
  )
  return pl.kernel(body, out_type=out_type, mesh=mesh, scratch_types=scratch,
                   compiler_params=pltpu.CompilerParams(
                       use_tc_tiling_on_sc=False),
                   name="seg_sum_cnt" if with_counts else "seg_sum")


HS = RPT // 2


def _newton_rsqrt(v):
  i = lax.bitcast_convert_type(v, jnp.int32)
  y = lax.bitcast_convert_type(jnp.int32(0x5F3759DF) - (i >> 1), jnp.float32)
  for _ in range(3):
    y = y * (1.5 - 0.5 * v * y * y)
  return y


_GDN = lax.GatherDimensionNumbers(
    offset_dims=(), collapsed_slice_dims=(0,), start_index_map=(0,))


def _lane_perm(v, idx):
  return lax.gather(v, idx[:, None], _GDN, slice_sizes=(1,),
                    mode=lax.GatherScatterMode.PROMISE_IN_BOUNDS)


def _lane_sum(v):
  lanes = lax.iota(jnp.int32, 16)
  for sh in (8, 4, 2, 1):
    v = v + _lane_perm(v, lanes ^ sh)
  return v


@functools.cache
def _seg2_builder():
  out_type = [
      jax.ShapeDtypeStruct((NC, N_PAD, D_HID), jnp.float32),
      jax.ShapeDtypeStruct((NC, N_PAD, D_HID), jnp.float32),
  ]
  scratch = [
      [pltpu.VMEM((CHUNK,), jnp.int32)] * 2,
      [pltpu.VMEM((CHUNK,), jnp.int32)] * 2,
      [pltpu.VMEM((CHUNK, D_HID), jnp.float32)] * 2,
      pltpu.VMEM_SHARED((N_PAD, D_HID), jnp.float32),
      [pltpu.SemaphoreType.DMA] * 2,
      [pltpu.VMEM((HS, D_HID), jnp.float32)] * 6,
      pltpu.VMEM((D_HID,), jnp.float32),
  ]

  def body(sums1, cnts1, r_hbm, b1_hbm, ei_hbm, zeros_hbm,
           sums_out, h_out, src_v, dst_v, rows_v, acc_sh, sem, ep, b_v):
    c = lax.axis_index("c")
    s = lax.axis_index("s")
    wid = s * NC + c
    rs = s * RPT
    pltpu.sync_copy(zeros_hbm.at[pl.ds(rs, RPT)], acc_sh.at[pl.ds(rs, RPT)])
    pltpu.sync_copy(b1_hbm, b_v)
    bias = b_v[...]
    sA, sB, cA, cB, rbuf, hbuf = ep

    for half in range(2):
      r0 = rs + half * HS
      pltpu.sync_copy(sums1.at[0, pl.ds(r0, HS)], sA)
      pltpu.sync_copy(sums1.at[1, pl.ds(r0, HS)], sB)
      pltpu.sync_copy(cnts1.at[0, pl.ds(r0, HS)], cA)
      pltpu.sync_copy(cnts1.at[1, pl.ds(r0, HS)], cB)
      pltpu.sync_copy(r_hbm.at[pl.ds(r0, HS)], rbuf)

      def row(j, _):
        cnt = jnp.maximum(cA[j, :] + cB[j, :], 1.0)
        t = (sA[j, :] + sB[j, :]) / cnt + bias + rbuf[j, :]
        n2 = jnp.maximum(_lane_sum(t * t), 1e-24)
        hbuf[j, :] = jnp.maximum(t * _newton_rsqrt(n2), 0.0)
        return 0

      lax.fori_loop(0, HS, row, 0)
      pltpu.sync_copy(hbuf, h_out.at[c, pl.ds(r0, HS)])
    plsc.subcore_barrier()

    def start_chunk(i):
      b = wid * EPW + i * CHUNK
      k = i % 2
      pltpu.sync_copy(ei_hbm.at[0, pl.ds(b, CHUNK)], src_v[k])
      pltpu.sync_copy(ei_hbm.at[1, pl.ds(b, CHUNK)], dst_v[k])
      return pltpu.async_copy(h_out.at[c].at[src_v[k]], rows_v[k], sem[k])

    gather = start_chunk(0)
    for i in range(NCHUNKS):
      k = i % 2
      nxt = start_chunk(i + 1) if i + 1 < NCHUNKS else None
      gather.wait()
      pltpu.sync_copy(rows_v[k], acc_sh.at[dst_v[k]], add=True)
      gather = nxt
    plsc.subcore_barrier()
    pltpu.sync_copy(acc_sh.at[pl.ds(rs, RPT)], sums_out.at[c, pl.ds(rs, RPT)])

  mesh = plsc.VectorSubcoreMesh(
      core_axis_name="c", subcore_axis_name="s", num_cores=NC, num_subcores=NS
  )
  return pl.kernel(body, out_type=out_type, mesh=mesh, scratch_types=scratch,
                   compiler_params=pltpu.CompilerParams(
                       use_tc_tiling_on_sc=False),
                   name="seg2_fused")


NR = N_NODES // 8
NRP = N_PAD // 8
PW = 8 * D_HID
PO = 8 * D_OUT


def _proj_body(x_ref, wl_ref, wr_ref, p_ref, r_ref):
  x = x_ref[...]
  p_ref[...] = jnp.dot(x, wl_ref[...], preferred_element_type=jnp.float32)
  r_ref[pl.ds(0, NR), :] = jnp.dot(x, wr_ref[...],
                                   preferred_element_type=jnp.float32)
  r_ref[pl.ds(NR, NRP - NR), :] = jnp.zeros((NRP - NR, PW), jnp.float32)


_proj = pl.pallas_call(
    _proj_body,
    out_shape=[jax.ShapeDtypeStruct((NR, PW), jnp.float32),
               jax.ShapeDtypeStruct((NRP, PW), jnp.float32)],
)


def _post2_body(sums, cnts, h, wl, wr, g, b2, o_ref):
  cnt = jnp.maximum(cnts[0] + cnts[1], 1.0)
  mean2 = (sums[0] + sums[1]) / cnt
  t = (jnp.dot(mean2, wl[...], preferred_element_type=jnp.float32)
       + jnp.dot(h[0], wr[...], preferred_element_type=jnp.float32)
       + b2[...])
  nrm = jnp.sqrt(jnp.dot(t * t, g[...], preferred_element_type=jnp.float32))
  inv = 1.0 / jnp.maximum(nrm, 1e-12)
  bc = lax.dot_general(inv, g[...], (((1,), (1,)), ((), ())),
                       preferred_element_type=jnp.float32)
  o_ref[...] = t * bc


_post2 = pl.pallas_call(
    _post2_body,
    out_shape=jax.ShapeDtypeStruct((NRP, PO), jnp.float32),
)


@jax.jit
def kernel(x, edge_index, Wl1, Wr1, b1, Wl2, Wr2, b2):
  ei = edge_index.astype(jnp.int32)
  zeros = jnp.zeros((N_PAD, D_HID), jnp.float32)
  ones = jnp.ones((CHUNK, D_HID), jnp.float32)
  eye8 = jnp.eye(8, dtype=jnp.float32)
  wl1_bd = jnp.kron(eye8, Wl1)
  wr1_bd = jnp.kron(eye8, Wr1)
  wl2_bd = jnp.kron(eye8, Wl2)
  wr2_bd = jnp.kron(eye8, Wr2)
  g_mat = jnp.kron(eye8, jnp.ones((D_OUT, 1), jnp.float32))
  b2_t = jnp.tile(b2, 8)[None]

  xp = x.reshape(NR, 8 * D_IN)
  pp, rp = _proj(xp, wl1_bd, wr1_bd)
  p = pp.reshape(N_NODES, D_HID)
  r = rp.reshape(N_PAD, D_HID)
  sums1, cnts1 = _seg_sum_builder(True)(p, ei, zeros, ones)
  sums2, h = _seg2_builder()(sums1, cnts1, r, b1, ei, zeros)
  sp = sums2.reshape(NC, NRP, PW)
  cp = cnts1.reshape(NC, NRP, PW)
  hp = h.reshape(NC, NRP, PW)
  op = _post2(sp, cp, hp, wl2_bd, wr2_bd, g_mat, b2_t)
  return op.reshape(N_PAD, D_OUT)[:N_NODES]


# speedup vs baseline: 33.1528x; 1.1374x over previous
import functools

import jax
import jax.numpy as jnp
from jax import lax
from jax.experimental import pallas as pl
from jax.experimental.pallas import tpu as pltpu
from jax.experimental.pallas import tpu_sc as plsc

N_NODES = 10000
N_EDGES = 320000
D_IN = 128
D_HID = 16
D_OUT = 128

NC, NS = 2, 16
NW = NC * NS
EPW = N_EDGES // NW
CHUNK = 2000
NCHUNKS = EPW // CHUNK
N_PAD = 10240
RPT = N_PAD // NS

@functools.cache
def _seg_sum_builder(with_counts):
  n_out = 2 if with_counts else 1
  out_type = [jax.ShapeDtypeStruct((NC, N_PAD, D_HID), jnp.float32)] * n_out
  scratch = [
      [pltpu.VMEM((CHUNK,), jnp.int32)] * 2,
      [pltpu.VMEM((CHUNK,), jnp.int32)] * 2,
      [pltpu.VMEM((CHUNK, D_HID), jnp.float32)] * 2,
      pltpu.VMEM_SHARED((N_PAD, D_HID), jnp.float32),
      [pltpu.SemaphoreType.DMA] * 2,
  ]
  if with_counts:
    scratch += [
        pltpu.VMEM((CHUNK, D_HID), jnp.float32),
        pltpu.VMEM_SHARED((N_PAD, D_HID), jnp.float32),
    ]

  def body(*refs):
    if with_counts:
      (tbl, ei_hbm, zeros_hbm, ones_hbm,
       sums_out, cnts_out, src_v, dst_v, rows_v, acc_sh, sem,
       ones_v, cacc_sh) = refs
    else:
      (tbl, ei_hbm, zeros_hbm,
       sums_out, src_v, dst_v, rows_v, acc_sh, sem) = refs
    c = lax.axis_index("c")
    s = lax.axis_index("s")
    wid = s * NC + c
    rs = s * RPT
    pltpu.sync_copy(zeros_hbm.at[pl.ds(rs, RPT)], acc_sh.at[pl.ds(rs, RPT)])
    if with_counts:
      pltpu.sync_copy(zeros_hbm.at[pl.ds(rs, RPT)], cacc_sh.at[pl.ds(rs, RPT)])
      pltpu.sync_copy(ones_hbm, ones_v)
    plsc.subcore_barrier()

    def start_chunk(i):
      b = wid * EPW + i * CHUNK
      k = i % 2
      pltpu.sync_copy(ei_hbm.at[0, pl.ds(b, CHUNK)], src_v[k])
      pltpu.sync_copy(ei_hbm.at[1, pl.ds(b, CHUNK)], dst_v[k])
      return pltpu.async_copy(tbl.at[src_v[k]], rows_v[k], sem[k])

    gather = start_chunk(0)
    for i in range(NCHUNKS):
      k = i % 2
      nxt = start_chunk(i + 1) if i + 1 < NCHUNKS else None
      gather.wait()
      pltpu.sync_copy(rows_v[k], acc_sh.at[dst_v[k]], add=True)
      if with_counts:
        pltpu.sync_copy(ones_v, cacc_sh.at[dst_v[k]], add=True)
      gather = nxt
    plsc.subcore_barrier()
    pltpu.sync_copy(acc_sh.at[pl.ds(rs, RPT)],
                    sums_out.at[c, pl.ds(rs, RPT)])
    if with_counts:
      pltpu.sync_copy(cacc_sh.at[pl.ds(rs, RPT)],
                      cnts_out.at[c, pl.ds(rs, RPT)])

  mesh = plsc.VectorSubcoreMesh(
      core_axis_name="c", subcore_axis_name="s", num_cores=NC, num_subcores=NS
  )
  return pl.kernel(body, out_type=out_type, mesh=mesh, scratch_types=scratch,
                   compiler_params=pltpu.CompilerParams(
                       use_tc_tiling_on_sc=False),
                   name="seg_sum_cnt" if with_counts else "seg_sum")


HS = RPT // 2


def _newton_rsqrt(v):
  i = lax.bitcast_convert_type(v, jnp.int32)
  y = lax.bitcast_convert_type(jnp.int32(0x5F3759DF) - (i >> 1), jnp.float32)
  for _ in range(3):
    y = y * (1.5 - 0.5 * v * y * y)
  return y


_GDN = lax.GatherDimensionNumbers(
    offset_dims=(), collapsed_slice_dims=(0,), start_index_map=(0,))


def _lane_perm(v, idx):
  return lax.gather(v, idx[:, None], _GDN, slice_sizes=(1,),
                    mode=lax.GatherScatterMode.PROMISE_IN_BOUNDS)


def _lane_sum(v):
  lanes = lax.iota(jnp.int32, 16)
  for sh in (8, 4, 2, 1):
    v = v + _lane_perm(v, lanes ^ sh)
  return v


@functools.cache
def _seg2_builder():
  out_type = [
      jax.ShapeDtypeStruct((NC, N_PAD, D_HID), jnp.float32),
      jax.ShapeDtypeStruct((NC, N_PAD, D_HID), jnp.float32),
  ]
  scratch = [
      [pltpu.VMEM((CHUNK,), jnp.int32)] * 2,
      [pltpu.VMEM((CHUNK,), jnp.int32)] * 2,
      [pltpu.VMEM((CHUNK, D_HID), jnp.float32)] * 2,
      pltpu.VMEM_SHARED((N_PAD, D_HID), jnp.float32),
      [pltpu.SemaphoreType.DMA] * 2,
      [pltpu.VMEM((HS, D_HID), jnp.float32)] * 6,
      pltpu.VMEM((D_HID,), jnp.float32),
  ]

  def body(sums1, cnts1, r_hbm, b1_hbm, ei_hbm, zeros_hbm,
           sums_out, h_out, src_v, dst_v, rows_v, acc_sh, sem, ep, b_v):
    c = lax.axis_index("c")
    s = lax.axis_index("s")
    wid = s * NC + c
    rs = s * RPT
    pltpu.sync_copy(zeros_hbm.at[pl.ds(rs, RPT)], acc_sh.at[pl.ds(rs, RPT)])
    pltpu.sync_copy(b1_hbm, b_v)
    bias = b_v[...]
    sA, sB, cA, cB, rbuf, hbuf = ep

    for half in range(2):
      r0 = rs + half * HS
      pltpu.sync_copy(sums1.at[0, pl.ds(r0, HS)], sA)
      pltpu.sync_copy(sums1.at[1, pl.ds(r0, HS)], sB)
      pltpu.sync_copy(cnts1.at[0, pl.ds(r0, HS)], cA)
      pltpu.sync_copy(cnts1.at[1, pl.ds(r0, HS)], cB)
      pltpu.sync_copy(r_hbm.at[pl.ds(r0, HS)], rbuf)

      def row(j, _):
        cnt = jnp.maximum(cA[j, :] + cB[j, :], 1.0)
        t = (sA[j, :] + sB[j, :]) / cnt + bias + rbuf[j, :]
        n2 = jnp.maximum(_lane_sum(t * t), 1e-24)
        hbuf[j, :] = jnp.maximum(t * _newton_rsqrt(n2), 0.0)
        return 0

      lax.fori_loop(0, HS, row, 0)
      pltpu.sync_copy(hbuf, h_out.at[c, pl.ds(r0, HS)])
    plsc.subcore_barrier()

    def start_chunk(i):
      b = wid * EPW + i * CHUNK
      k = i % 2
      pltpu.sync_copy(ei_hbm.at[0, pl.ds(b, CHUNK)], src_v[k])
      pltpu.sync_copy(ei_hbm.at[1, pl.ds(b, CHUNK)], dst_v[k])
      return pltpu.async_copy(h_out.at[c].at[src_v[k]], rows_v[k], sem[k])

    gather = start_chunk(0)
    for i in range(NCHUNKS):
      k = i % 2
      nxt = start_chunk(i + 1) if i + 1 < NCHUNKS else None
      gather.wait()
      pltpu.sync_copy(rows_v[k], acc_sh.at[dst_v[k]], add=True)
      gather = nxt
    plsc.subcore_barrier()
    pltpu.sync_copy(acc_sh.at[pl.ds(rs, RPT)], sums_out.at[c, pl.ds(rs, RPT)])

  mesh = plsc.VectorSubcoreMesh(
      core_axis_name="c", subcore_axis_name="s", num_cores=NC, num_subcores=NS
  )
  return pl.kernel(body, out_type=out_type, mesh=mesh, scratch_types=scratch,
                   compiler_params=pltpu.CompilerParams(
                       use_tc_tiling_on_sc=False),
                   name="seg2_fused")


NR = N_NODES // 8
NRP = N_PAD // 8
PW = 8 * D_HID
PO = 8 * D_OUT


def _block_diag8(w):
  a, b = w.shape
  t = jnp.tile(w, (8, 8))
  rg = lax.broadcasted_iota(jnp.int32, (8 * a, 8 * b), 0) // a
  cg = lax.broadcasted_iota(jnp.int32, (8 * a, 8 * b), 1) // b
  return jnp.where(rg == cg, t, 0.0)


def _proj_body(x_ref, wl_ref, wr_ref, p_ref, r_ref):
  x = x_ref[...].reshape(NR, 8 * D_IN)
  wl = _block_diag8(wl_ref[...])
  wr = _block_diag8(wr_ref[...])
  p_ref[...] = jnp.dot(x, wl, preferred_element_type=jnp.float32)
  r_ref[pl.ds(0, NR), :] = jnp.dot(x, wr, preferred_element_type=jnp.float32)
  r_ref[pl.ds(NR, NRP - NR), :] = jnp.zeros((NRP - NR, PW), jnp.float32)


_proj = pl.pallas_call(
    _proj_body,
    out_shape=[jax.ShapeDtypeStruct((NR, PW), jnp.float32),
               jax.ShapeDtypeStruct((NRP, PW), jnp.float32)],
)


def _post2_body(sums, cnts, h, wl_ref, wr_ref, b2, o_ref):
  wl = _block_diag8(wl_ref[...])
  wr = _block_diag8(wr_ref[...])
  g = _block_diag8(jnp.ones((D_OUT, 1), jnp.float32))
  b2t = jnp.tile(b2[...], (1, 8))
  cnt = jnp.maximum(cnts[0] + cnts[1], 1.0)
  mean2 = (sums[0] + sums[1]) / cnt
  t = (jnp.dot(mean2, wl, preferred_element_type=jnp.float32)
       + jnp.dot(h[0], wr, preferred_element_type=jnp.float32)
       + b2t)
  nrm = jnp.sqrt(jnp.dot(t * t, g, preferred_element_type=jnp.float32))
  inv = 1.0 / jnp.maximum(nrm, 1e-12)
  bc = lax.dot_general(inv, g, (((1,), (1,)), ((), ())),
                       preferred_element_type=jnp.float32)
  o_ref[...] = (t * bc)[:NR].reshape(N_NODES, D_OUT)


_post2 = pl.pallas_call(
    _post2_body,
    out_shape=jax.ShapeDtypeStruct((N_NODES, D_OUT), jnp.float32),
)


@jax.jit
def kernel(x, edge_index, Wl1, Wr1, b1, Wl2, Wr2, b2):
  ei = edge_index.astype(jnp.int32)
  zeros = jnp.zeros((N_PAD, D_HID), jnp.float32)
  ones = jnp.ones((CHUNK, D_HID), jnp.float32)

  pp, rp = _proj(x, Wl1, Wr1)
  p = pp.reshape(N_NODES, D_HID)
  r = rp.reshape(N_PAD, D_HID)
  sums1, cnts1 = _seg_sum_builder(True)(p, ei, zeros, ones)
  sums2, h = _seg2_builder()(sums1, cnts1, r, b1, ei, zeros)
  sp = sums2.reshape(NC, NRP, PW)
  cp = cnts1.reshape(NC, NRP, PW)
  hp = h.reshape(NC, NRP, PW)
  return _post2(sp, cp, hp, Wl2, Wr2, b2.reshape(1, D_OUT))
